# baseline (speedup 1.0000x reference)
import jax
import jax.numpy as jnp
import numpy as np
from jax.experimental import pallas as pl
from jax.experimental.pallas import tpu as pltpu

B = 1024
NINP = 512
NHID = 1024
NBLK = 16
TOPK = 8
BLK = NHID // NBLK
ATT = 4 * BLK
GIN = ATT * NBLK
DK = 64
GT = 256
NG = 3 * NHID // GT
NOUT = NHID // GT


def _body(inp_ref, hx_ref, p0_ref, p1_ref, wvs_ref, wih_ref, whh_ref,
          bih_ref, bhh_ref,
          hx_out_ref, mask_out_ref,
          att_sc, hxb_sc, gates_sc, m16_sc):
    s = pl.program_id(0)

    @pl.when(s == 0)
    def _prep():
        hxb_sc[...] = hx_ref[...].astype(jnp.bfloat16)
        vv1 = jnp.dot(inp_ref[...], wvs_ref[...],
                      preferred_element_type=jnp.float32)
        p1 = p1_ref[...]
        for blk in range(NBLK):
            att_sc[:, blk * ATT:(blk + 1) * ATT] = (
                p1[:, blk:blk + 1] * vv1).astype(jnp.bfloat16)
        p0 = p0_ref[...]
        colid = jax.lax.broadcasted_iota(jnp.int32, (B, NBLK), 1)
        mcols = []
        for i in range(NBLK):
            vi = p0[:, i:i + 1]
            gt = jnp.sum(jnp.where(p0 > vi, 1.0, 0.0), axis=1, keepdims=True)
            eqb = jnp.sum(jnp.where((p0 == vi) & (colid < i), 1.0, 0.0),
                          axis=1, keepdims=True)
            mcols.append(jnp.where(gt + eqb >= float(NBLK - TOPK), 1.0, 0.0))
        m16_sc[...] = jnp.concatenate(mcols, axis=1)

    dn = (((1,), (1,)), ((), ()))
    gih = jax.lax.dot_general(att_sc[...],
                              wih_ref[...].astype(jnp.bfloat16), dn,
                              preferred_element_type=jnp.float32)
    ghh = jax.lax.dot_general(hxb_sc[...],
                              whh_ref[...].astype(jnp.bfloat16), dn,
                              preferred_element_type=jnp.float32)

    @pl.when(s < NG - NOUT)
    def _stash():
        gates_sc[:, pl.ds(s * GT, GT)] = (gih + ghh).astype(jnp.bfloat16)

    @pl.when(s >= NG - NOUT)
    def _finish():
        t = s - (NG - NOUT)
        c0 = t * GT

        def pre(g):
            gsum = gates_sc[:, pl.ds(g * NHID + c0, GT)].astype(jnp.float32)
            bi = bih_ref[0:1, pl.ds(g * NHID + c0, GT)]
            bh = bhh_ref[0:1, pl.ds(g * NHID + c0, GT)]
            return gsum + (bi + bh)

        r = jax.nn.sigmoid(pre(0))
        z = jax.nn.sigmoid(pre(1))
        gi_n = gih + bih_ref[0:1, pl.ds(2 * NHID + c0, GT)]
        gh_n = ghh + bhh_ref[0:1, pl.ds(2 * NHID + c0, GT)]
        n = jnp.tanh(gi_n + r * gh_n)
        hxt = hx_ref[:, pl.ds(c0, GT)]
        hx_new = (1.0 - z) * n + z * hxt
        erow = jax.lax.broadcasted_iota(jnp.int32, (NBLK, GT), 0)
        ecol = jax.lax.broadcasted_iota(jnp.int32, (NBLK, GT), 1) // BLK
        expand = jnp.where(erow == ecol + t * (GT // BLK), 1.0, 0.0)
        mt = jnp.dot(m16_sc[...], expand, preferred_element_type=jnp.float32)
        hx_out_ref[:, pl.ds(c0, GT)] = mt * hx_new + (1.0 - mt) * hxt
        mask_out_ref[:, pl.ds(c0, GT)] = mt


def kernel(inp, hx, w_qs, w_ks, w_vs, W_ih, W_hh, b_ih, b_hh, step):
    del step
    b = inp.shape[0]
    wq_bd = jnp.concatenate(
        [jnp.concatenate(
            [jnp.zeros((BLK, nb * BLK), jnp.float32), w_qs[nb],
             jnp.zeros((BLK, NHID - (nb + 1) * BLK), jnp.float32)], axis=1)
         for nb in range(NBLK)], axis=0)
    q = (hx @ wq_bd).reshape(b, NBLK, BLK)
    kk1 = inp @ w_ks[1]
    kk = jnp.concatenate([jnp.zeros_like(kk1[:, None, :]), kk1[:, None, :]],
                         axis=1)
    s1 = (jnp.einsum('bqd,bkd->bqk', q, kk) / np.sqrt(DK))[:, :, 1]
    m = jnp.maximum(s1, 0.0)
    e0 = jnp.exp(-m)
    e1 = jnp.exp(s1 - m)
    den = e0 + e1
    p0 = e0 / den
    p1 = e1 / den

    wvs1 = w_vs[1]
    bih2 = b_ih.reshape(1, 3 * NHID)
    bhh2 = b_hh.reshape(1, 3 * NHID)

    in_specs = [
        pl.BlockSpec((B, NINP), lambda s: (0, 0)),
        pl.BlockSpec((B, NHID), lambda s: (0, 0)),
        pl.BlockSpec((B, NBLK), lambda s: (0, 0)),
        pl.BlockSpec((B, NBLK), lambda s: (0, 0)),
        pl.BlockSpec((NINP, ATT), lambda s: (0, 0)),
        pl.BlockSpec((GT, GIN), lambda s: (s, 0)),
        pl.BlockSpec((GT, NHID), lambda s: (s, 0)),
        pl.BlockSpec((1, 3 * NHID), lambda s: (0, 0)),
        pl.BlockSpec((1, 3 * NHID), lambda s: (0, 0)),
    ]
    out_specs = [
        pl.BlockSpec((B, NHID), lambda s: (0, 0)),
        pl.BlockSpec((B, NHID), lambda s: (0, 0)),
    ]
    hx_out, mask = pl.pallas_call(
        _body,
        grid=(NG,),
        in_specs=in_specs,
        out_specs=out_specs,
        out_shape=[
            jax.ShapeDtypeStruct((B, NHID), jnp.float32),
            jax.ShapeDtypeStruct((B, NHID), jnp.float32),
        ],
        scratch_shapes=[
            pltpu.VMEM((B, GIN), jnp.bfloat16),
            pltpu.VMEM((B, NHID), jnp.bfloat16),
            pltpu.VMEM((B, 2 * NHID), jnp.bfloat16),
            pltpu.VMEM((B, NBLK), jnp.float32),
        ],
        compiler_params=pltpu.CompilerParams(
            dimension_semantics=("arbitrary",),
            vmem_limit_bytes=63 * 1024 * 1024,
        ),
    )(inp, hx, p0, p1, wvs1, W_ih, W_hh, bih2, bhh2)
    return hx_out, mask


# speedup vs baseline: 1.0474x; 1.0474x over previous
import jax
import jax.numpy as jnp
import numpy as np
from jax.experimental import pallas as pl
from jax.experimental.pallas import tpu as pltpu

B = 1024
NINP = 512
NHID = 1024
NBLK = 16
TOPK = 8
BLK = NHID // NBLK
ATT = 4 * BLK
GIN = ATT * NBLK
DK = 64
GT = 256
NG = 3 * NHID // GT
NOUT = NHID // GT


def _body(inp_ref, hx_ref, p0_ref, p1_ref, wvs_ref, wih_ref, whh_ref,
          bih_ref, bhh_ref,
          hx_out_ref, mask_out_ref,
          att_sc, hxb_sc, gates_sc, m16_sc):
    s = pl.program_id(0)

    @pl.when(s == 0)
    def _prep():
        hxb_sc[...] = hx_ref[...].astype(jnp.bfloat16)
        vv1 = jnp.dot(inp_ref[...], wvs_ref[...],
                      preferred_element_type=jnp.float32)
        p1 = p1_ref[...]
        for blk in range(NBLK):
            att_sc[:, blk * ATT:(blk + 1) * ATT] = (
                p1[:, blk:blk + 1] * vv1).astype(jnp.bfloat16)
        p0 = p0_ref[...]
        pj = jnp.concatenate([p0] * NBLK, axis=1)
        pi = jnp.repeat(p0, NBLK, axis=1)
        cc = jax.lax.broadcasted_iota(jnp.int32, (B, NBLK * NBLK), 1)
        jj = cc % NBLK
        ii = cc // NBLK
        cnt = jnp.where((pj > pi) | ((pj == pi) & (jj < ii)), 1.0, 0.0)
        gsum = jnp.where(
            jax.lax.broadcasted_iota(jnp.int32, (NBLK * NBLK, NBLK), 0)
            // NBLK
            == jax.lax.broadcasted_iota(jnp.int32, (NBLK * NBLK, NBLK), 1),
            1.0, 0.0)
        rank = jnp.dot(cnt, gsum, preferred_element_type=jnp.float32)
        m16_sc[...] = jnp.where(rank >= float(NBLK - TOPK), 1.0, 0.0)

    dn = (((1,), (1,)), ((), ()))
    gih = jax.lax.dot_general(att_sc[...],
                              wih_ref[...].astype(jnp.bfloat16), dn,
                              preferred_element_type=jnp.float32)
    ghh = jax.lax.dot_general(hxb_sc[...],
                              whh_ref[...].astype(jnp.bfloat16), dn,
                              preferred_element_type=jnp.float32)

    @pl.when(s < NG - NOUT)
    def _stash():
        gates_sc[:, pl.ds(s * GT, GT)] = (gih + ghh).astype(jnp.bfloat16)

    @pl.when(s >= NG - NOUT)
    def _finish():
        t = s - (NG - NOUT)
        c0 = t * GT

        def pre(g):
            gsum = gates_sc[:, pl.ds(g * NHID + c0, GT)].astype(jnp.float32)
            bi = bih_ref[0:1, pl.ds(g * NHID + c0, GT)]
            bh = bhh_ref[0:1, pl.ds(g * NHID + c0, GT)]
            return gsum + (bi + bh)

        r = jax.nn.sigmoid(pre(0))
        z = jax.nn.sigmoid(pre(1))
        gi_n = gih + bih_ref[0:1, pl.ds(2 * NHID + c0, GT)]
        gh_n = ghh + bhh_ref[0:1, pl.ds(2 * NHID + c0, GT)]
        n = jnp.tanh(gi_n + r * gh_n)
        hxt = hx_ref[:, pl.ds(c0, GT)]
        hx_new = (1.0 - z) * n + z * hxt
        erow = jax.lax.broadcasted_iota(jnp.int32, (NBLK, GT), 0)
        ecol = jax.lax.broadcasted_iota(jnp.int32, (NBLK, GT), 1) // BLK
        expand = jnp.where(erow == ecol + t * (GT // BLK), 1.0, 0.0)
        mt = jnp.dot(m16_sc[...], expand, preferred_element_type=jnp.float32)
        hx_out_ref[:, pl.ds(c0, GT)] = mt * hx_new + (1.0 - mt) * hxt
        mask_out_ref[:, pl.ds(c0, GT)] = mt


def kernel(inp, hx, w_qs, w_ks, w_vs, W_ih, W_hh, b_ih, b_hh, step):
    del step
    b = inp.shape[0]
    q = jnp.einsum('bnd,nde->bne', hx.reshape(b, NBLK, BLK), w_qs)
    kk1 = inp @ w_ks[1]
    kk = jnp.concatenate([jnp.zeros_like(kk1[:, None, :]), kk1[:, None, :]],
                         axis=1)
    s1 = (jnp.einsum('bqd,bkd->bqk', q, kk) / np.sqrt(DK))[:, :, 1]
    m = jnp.maximum(s1, 0.0)
    e0 = jnp.exp(-m)
    e1 = jnp.exp(s1 - m)
    den = e0 + e1
    p0 = e0 / den
    p1 = e1 / den

    wvs1 = w_vs[1]
    bih2 = b_ih.reshape(1, 3 * NHID)
    bhh2 = b_hh.reshape(1, 3 * NHID)

    in_specs = [
        pl.BlockSpec((B, NINP), lambda s: (0, 0)),
        pl.BlockSpec((B, NHID), lambda s: (0, 0)),
        pl.BlockSpec((B, NBLK), lambda s: (0, 0)),
        pl.BlockSpec((B, NBLK), lambda s: (0, 0)),
        pl.BlockSpec((NINP, ATT), lambda s: (0, 0)),
        pl.BlockSpec((GT, GIN), lambda s: (s, 0)),
        pl.BlockSpec((GT, NHID), lambda s: (s, 0)),
        pl.BlockSpec((1, 3 * NHID), lambda s: (0, 0)),
        pl.BlockSpec((1, 3 * NHID), lambda s: (0, 0)),
    ]
    out_specs = [
        pl.BlockSpec((B, NHID), lambda s: (0, 0)),
        pl.BlockSpec((B, NHID), lambda s: (0, 0)),
    ]
    hx_out, mask = pl.pallas_call(
        _body,
        grid=(NG,),
        in_specs=in_specs,
        out_specs=out_specs,
        out_shape=[
            jax.ShapeDtypeStruct((B, NHID), jnp.float32),
            jax.ShapeDtypeStruct((B, NHID), jnp.float32),
        ],
        scratch_shapes=[
            pltpu.VMEM((B, GIN), jnp.bfloat16),
            pltpu.VMEM((B, NHID), jnp.bfloat16),
            pltpu.VMEM((B, 2 * NHID), jnp.bfloat16),
            pltpu.VMEM((B, NBLK), jnp.float32),
        ],
        compiler_params=pltpu.CompilerParams(
            dimension_semantics=("arbitrary",),
            vmem_limit_bytes=63 * 1024 * 1024,
        ),
    )(inp, hx, p0, p1, wvs1, W_ih, W_hh, bih2, bhh2)
    return hx_out, mask


# speedup vs baseline: 1.0629x; 1.0149x over previous
import jax
import jax.numpy as jnp
import numpy as np
from jax.experimental import pallas as pl
from jax.experimental.pallas import tpu as pltpu

B = 1024
NINP = 512
NHID = 1024
NBLK = 16
TOPK = 8
BLK = NHID // NBLK
ATT = 4 * BLK
GIN = ATT * NBLK
DK = 64
GT = 256
NG = 3 * NHID // GT
NOUT = NHID // GT


def _body(inp_ref, hx_ref, p0_ref, p1_ref, wvs_ref, wih_ref, whh_ref,
          bih_ref, bhh_ref,
          hx_out_ref, mask_out_ref,
          att_sc, hxb_sc, gates_sc, m16_sc):
    s = pl.program_id(0)

    @pl.when(s == 0)
    def _prep():
        hxb_sc[...] = hx_ref[...].astype(jnp.bfloat16)
        vv1 = jnp.dot(inp_ref[...], wvs_ref[...],
                      preferred_element_type=jnp.float32)
        p1 = p1_ref[...]
        for blk in range(NBLK):
            att_sc[:, blk * ATT:(blk + 1) * ATT] = (
                p1[:, blk:blk + 1] * vv1).astype(jnp.bfloat16)
        p0 = p0_ref[...]
        pj = jnp.concatenate([p0] * NBLK, axis=1)
        pi = jnp.repeat(p0, NBLK, axis=1)
        cc = jax.lax.broadcasted_iota(jnp.int32, (B, NBLK * NBLK), 1)
        jj = cc % NBLK
        ii = cc // NBLK
        cnt = jnp.where((pj > pi) | ((pj == pi) & (jj < ii)), 1.0, 0.0)
        gsum = jnp.where(
            jax.lax.broadcasted_iota(jnp.int32, (NBLK * NBLK, NBLK), 0)
            // NBLK
            == jax.lax.broadcasted_iota(jnp.int32, (NBLK * NBLK, NBLK), 1),
            1.0, 0.0)
        rank = jnp.dot(cnt, gsum, preferred_element_type=jnp.float32)
        m16_sc[...] = jnp.where(rank >= float(NBLK - TOPK), 1.0, 0.0)

    dn = (((1,), (1,)), ((), ()))
    gih = jax.lax.dot_general(att_sc[...],
                              wih_ref[...].astype(jnp.bfloat16), dn,
                              preferred_element_type=jnp.float32)
    ghh = jax.lax.dot_general(hxb_sc[...],
                              whh_ref[...].astype(jnp.bfloat16), dn,
                              preferred_element_type=jnp.float32)

    @pl.when(s < NG - NOUT)
    def _stash():
        gates_sc[:, pl.ds(s * GT, GT)] = (gih + ghh).astype(jnp.bfloat16)

    @pl.when(s >= NG - NOUT)
    def _finish():
        t = s - (NG - NOUT)
        c0 = t * GT

        def pre(g):
            gsum = gates_sc[:, pl.ds(g * NHID + c0, GT)].astype(jnp.float32)
            bi = bih_ref[0:1, pl.ds(g * NHID + c0, GT)]
            bh = bhh_ref[0:1, pl.ds(g * NHID + c0, GT)]
            return gsum + (bi + bh)

        r = jax.nn.sigmoid(pre(0))
        z = jax.nn.sigmoid(pre(1))
        gi_n = gih + bih_ref[0:1, pl.ds(2 * NHID + c0, GT)]
        gh_n = ghh + bhh_ref[0:1, pl.ds(2 * NHID + c0, GT)]
        n = jnp.tanh(gi_n + r * gh_n)
        hxt = hx_ref[:, pl.ds(c0, GT)]
        hx_new = (1.0 - z) * n + z * hxt
        erow = jax.lax.broadcasted_iota(jnp.int32, (NBLK, GT), 0)
        ecol = jax.lax.broadcasted_iota(jnp.int32, (NBLK, GT), 1) // BLK
        expand = jnp.where(erow == ecol + t * (GT // BLK), 1.0, 0.0)
        mt = jnp.dot(m16_sc[...], expand, preferred_element_type=jnp.float32)
        hx_out_ref[...] = mt * hx_new + (1.0 - mt) * hxt
        mask_out_ref[...] = mt


def kernel(inp, hx, w_qs, w_ks, w_vs, W_ih, W_hh, b_ih, b_hh, step):
    del step
    b = inp.shape[0]
    q = jnp.einsum('bnd,nde->bne', hx.reshape(b, NBLK, BLK), w_qs)
    kk1 = inp @ w_ks[1]
    kk = jnp.concatenate([jnp.zeros_like(kk1[:, None, :]), kk1[:, None, :]],
                         axis=1)
    s1 = (jnp.einsum('bqd,bkd->bqk', q, kk) / np.sqrt(DK))[:, :, 1]
    m = jnp.maximum(s1, 0.0)
    e0 = jnp.exp(-m)
    e1 = jnp.exp(s1 - m)
    den = e0 + e1
    p0 = e0 / den
    p1 = e1 / den

    wvs1 = w_vs[1]
    bih2 = b_ih.reshape(1, 3 * NHID)
    bhh2 = b_hh.reshape(1, 3 * NHID)

    in_specs = [
        pl.BlockSpec((B, NINP), lambda s: (0, 0)),
        pl.BlockSpec((B, NHID), lambda s: (0, 0)),
        pl.BlockSpec((B, NBLK), lambda s: (0, 0)),
        pl.BlockSpec((B, NBLK), lambda s: (0, 0)),
        pl.BlockSpec((NINP, ATT), lambda s: (0, 0)),
        pl.BlockSpec((GT, GIN), lambda s: (s, 0)),
        pl.BlockSpec((GT, NHID), lambda s: (s, 0)),
        pl.BlockSpec((1, 3 * NHID), lambda s: (0, 0)),
        pl.BlockSpec((1, 3 * NHID), lambda s: (0, 0)),
    ]
    out_specs = [
        pl.BlockSpec((B, GT), lambda s: (0, jnp.maximum(s - (NG - NOUT), 0))),
        pl.BlockSpec((B, GT), lambda s: (0, jnp.maximum(s - (NG - NOUT), 0))),
    ]
    hx_out, mask = pl.pallas_call(
        _body,
        grid=(NG,),
        in_specs=in_specs,
        out_specs=out_specs,
        out_shape=[
            jax.ShapeDtypeStruct((B, NHID), jnp.float32),
            jax.ShapeDtypeStruct((B, NHID), jnp.float32),
        ],
        scratch_shapes=[
            pltpu.VMEM((B, GIN), jnp.bfloat16),
            pltpu.VMEM((B, NHID), jnp.bfloat16),
            pltpu.VMEM((B, 2 * NHID), jnp.bfloat16),
            pltpu.VMEM((B, NBLK), jnp.float32),
        ],
        compiler_params=pltpu.CompilerParams(
            dimension_semantics=("arbitrary",),
            vmem_limit_bytes=63 * 1024 * 1024,
        ),
    )(inp, hx, p0, p1, wvs1, W_ih, W_hh, bih2, bhh2)
    return hx_out, mask
